# baseline (speedup 1.0000x reference)
import jax
import jax.numpy as jnp
from jax import lax
from jax.experimental import pallas as pl
from jax.experimental.pallas import tpu as pltpu
from jax.experimental.pallas import tpu_sc as plsc

N = 10000
E = 320000
F = 128
D = 4
G = 256

NC = 2
NS = 16
NW = NC * NS
EW = E // NW
BE = 16
NB = EW // BE
ZB = 64
NP = 10240
RPS = NP // NS

_LOG1P_C = (
    0.00014158017492754693, 0.995426661775425, -0.4640707011025748,
    0.21640858368174304, -0.05486231128931281,
)

BN = 400
EPK = 8
BEP = 400


def _proj1_body(x_ref, w_ref, b_ref, t_ref):
    t_ref[0] = jnp.dot(x_ref[...], w_ref[...],
                       preferred_element_type=jnp.float32) + b_ref[...]


def _proj2_body(x_ref, a_ref, w_ref, b_ref, h_ref, t_ref):
    h = x_ref[...] + a_ref[0] + a_ref[1]
    h_ref[...] = h
    t_ref[0] = jnp.dot(h, w_ref[...],
                       preferred_element_type=jnp.float32) + b_ref[...]


def _edge_proj_body(ea_ref, we_ref, e_ref):
    e_ref[...] = jnp.dot(ea_ref[...], we_ref[...],
                         preferred_element_type=jnp.float32)


def _pool_body(h_ref, a_ref, b_ref, wo_ref, bo_ref, o_ref, acc_ref):
    i = pl.program_id(0)

    @pl.when(i == 0)
    def _():
        acc_ref[...] = jnp.zeros_like(acc_ref)

    h2 = h_ref[...] + a_ref[0] + a_ref[1]
    oh = (b_ref[...] == lax.broadcasted_iota(jnp.int32, (BN, G), 1)
          ).astype(jnp.float32)
    acc_ref[...] += lax.dot_general(oh, h2, (((0,), (0,)), ((), ())),
                                    preferred_element_type=jnp.float32)

    @pl.when(i == pl.num_programs(0) - 1)
    def _():
        o_ref[...] = jnp.dot(acc_ref[...], wo_ref[...],
                             preferred_element_type=jnp.float32) + bo_ref[...]


def _node_proj1(x, w, b):
    return pl.pallas_call(
        _proj1_body,
        grid=(N // BN, 2),
        in_specs=[
            pl.BlockSpec((BN, F), lambda i, j: (i, 0)),
            pl.BlockSpec((F, 2 * F), lambda i, j: (0, j)),
            pl.BlockSpec((1, 2 * F), lambda i, j: (0, j)),
        ],
        out_specs=pl.BlockSpec((1, BN, 2 * F), lambda i, j: (j, i, 0)),
        out_shape=jax.ShapeDtypeStruct((2, NP, 2 * F), jnp.float32),
    )(x, w, b)


def _node_proj2(x, aggp, w, b):
    return pl.pallas_call(
        _proj2_body,
        grid=(N // BN, 2),
        in_specs=[
            pl.BlockSpec((BN, F), lambda i, j: (i, 0)),
            pl.BlockSpec((2, BN, F), lambda i, j: (0, i, 0)),
            pl.BlockSpec((F, 2 * F), lambda i, j: (0, j)),
            pl.BlockSpec((1, 2 * F), lambda i, j: (0, j)),
        ],
        out_specs=[
            pl.BlockSpec((BN, F), lambda i, j: (i, 0)),
            pl.BlockSpec((1, BN, 2 * F), lambda i, j: (j, i, 0)),
        ],
        out_shape=[
            jax.ShapeDtypeStruct((N, F), jnp.float32),
            jax.ShapeDtypeStruct((2, NP, 2 * F), jnp.float32),
        ],
    )(x, aggp, w, b)


def _edge_proj(ea8, wek):
    out = pl.pallas_call(
        _edge_proj_body,
        grid=(E // EPK // BEP,),
        in_specs=[
            pl.BlockSpec((BEP, EPK * D), lambda i: (i, 0)),
            pl.BlockSpec((EPK * D, EPK * 2 * F), lambda i: (0, 0)),
        ],
        out_specs=pl.BlockSpec((BEP, EPK * 2 * F), lambda i: (i, 0)),
        out_shape=jax.ShapeDtypeStruct((E // EPK, EPK * 2 * F), jnp.float32),
    )(ea8, wek)
    return out.reshape(E, 2 * F)


def _pool(h, aggp, batch2d, wo, bo):
    return pl.pallas_call(
        _pool_body,
        grid=(N // BN,),
        in_specs=[
            pl.BlockSpec((BN, F), lambda i: (i, 0)),
            pl.BlockSpec((2, BN, F), lambda i: (0, i, 0)),
            pl.BlockSpec((BN, 1), lambda i: (i, 0)),
            pl.BlockSpec((F, 1), lambda i: (0, 0)),
            pl.BlockSpec((1, 1), lambda i: (0, 0)),
        ],
        out_specs=pl.BlockSpec((G, 1), lambda i: (0, 0)),
        out_shape=jax.ShapeDtypeStruct((G, 1), jnp.float32),
        scratch_shapes=[pltpu.VMEM((G, F), jnp.float32)],
    )(h, aggp, batch2d, wo, bo)


def _msg_edge(g, geb, mbb, e):
    sls = [pl.ds(ch * 16, 16) for ch in range(8)]
    sl2s = [pl.ds(F + ch * 16, 16) for ch in range(8)]
    a = [g[e, s] + g[BE + e, s] + geb[e, s] for s in sls]
    b2 = [g[e, s] + g[BE + e, s] + geb[e, s] for s in sl2s]
    u = [jnp.exp(-x) for x in a]
    r = [1.0 / (1.0 + x) for x in u]
    t = [jnp.exp(jnp.minimum(x, -x)) for x in b2]
    lp = [jnp.full((16,), _LOG1P_C[-1], jnp.float32)] * 8
    for co in _LOG1P_C[-2::-1]:
        lp = [p * x + co for p, x in zip(lp, t)]
    sp = [jnp.maximum(x, 0.0) + p for x, p in zip(b2, lp)]
    for ch in range(8):
        mbb[e, sls[ch]] = r[ch] * sp[ch]


def _sc_edge_body(t2, ee, di, si, out,
                  comb0, comb1, sdx0, sdx1, gb0, gb1, ge0, ge1, mb0, mb1,
                  zb, acc,
                  semg0, semg1, seme0, seme1, semi0, semi1,
                  semx0, semx1, sems0, sems1):
    c = lax.axis_index("c")
    s = lax.axis_index("s")
    w = s * NC + c
    comb = (comb0, comb1)
    sdx = (sdx0, sdx1)
    gb = (gb0, gb1)
    ge = (ge0, ge1)
    mb = (mb0, mb1)
    semg = (semg0, semg1)
    seme = (seme0, seme1)
    semi = (semi0, semi1)
    semx = (semx0, semx1)
    sems = (sems0, sems1)

    def _zrow(e, carry):
        for ch in range(8):
            zb[e, pl.ds(ch * 16, 16)] = jnp.zeros((16,), jnp.float32)
        return carry

    lax.fori_loop(0, ZB, _zrow, 0)
    for j in range(RPS // ZB):
        pltpu.sync_copy(zb, acc.at[pl.ds(s * RPS + j * ZB, ZB)])
    plsc.subcore_barrier()

    def _ibase(b):
        return w * EW + b * BE

    def _issue_idx(b, sl):
        base = _ibase(b)
        pltpu.async_copy(di.at[pl.ds(base, BE)],
                         comb[sl].at[pl.ds(0, BE)], semi[sl])
        pltpu.async_copy(si.at[pl.ds(base, BE)],
                         comb[sl].at[pl.ds(BE, BE)], semi[sl])

    def _wait_idx(b, sl):
        base = _ibase(b)
        pltpu.make_async_copy(di.at[pl.ds(base, BE)],
                              comb[sl].at[pl.ds(0, BE)], semi[sl]).wait()
        pltpu.make_async_copy(si.at[pl.ds(base, BE)],
                              comb[sl].at[pl.ds(BE, BE)], semi[sl]).wait()
        comb[sl][pl.ds(BE, BE)] = comb[sl][pl.ds(BE, BE)] + NP

    def _issue_gather(b, sl):
        pltpu.async_copy(t2.at[comb[sl]], gb[sl], semg[sl])
        pltpu.async_copy(ee.at[pl.ds(_ibase(b), BE)], ge[sl], seme[sl])

    def _wait_gather(b, sl):
        pltpu.make_async_copy(t2.at[comb[sl]], gb[sl], semg[sl]).wait()
        pltpu.make_async_copy(ee.at[pl.ds(_ibase(b), BE)], ge[sl],
                              seme[sl]).wait()

    def _wait_scatter(sl):
        pltpu.make_async_copy(mb[sl], acc.at[sdx[sl]], sems[sl]).wait()

    pltpu.sync_copy(di.at[pl.ds(_ibase(0), BE)], comb0.at[pl.ds(0, BE)])
    pltpu.sync_copy(si.at[pl.ds(_ibase(0), BE)], comb0.at[pl.ds(BE, BE)])
    comb0[pl.ds(BE, BE)] = comb0[pl.ds(BE, BE)] + NP
    _issue_gather(0, 0)
    _issue_idx(1, 1)

    def _batch(b, sl):
        bt = jnp.int32(b)
        nsl = 1 - sl

        @pl.when(bt + 1 < NB)
        def _():
            _wait_idx(b + 1, nsl)
            _issue_gather(b + 1, nsl)

        _wait_gather(b, sl)

        @pl.when(bt + 2 < NB)
        def _():
            _issue_idx(b + 2, sl)

        @pl.when(bt >= 2)
        def _():
            _wait_scatter(sl)

        pltpu.async_copy(di.at[pl.ds(_ibase(b), BE)], sdx[sl], semx[sl])

        @plsc.parallel_loop(0, BE, unroll=4)
        def _edge(e):
            _msg_edge(gb[sl], ge[sl], mb[sl], e)

        pltpu.make_async_copy(di.at[pl.ds(_ibase(b), BE)], sdx[sl],
                              semx[sl]).wait()
        pltpu.async_copy(mb[sl], acc.at[sdx[sl]], sems[sl], add=True)

    def _super(o, carry):
        for sl in (0, 1):
            _batch(o * 2 + sl, sl)
        return carry

    lax.fori_loop(0, NB // 2, _super, 0)
    if NB % 2:
        _batch(NB - 1, 0)
    _wait_scatter(0)
    _wait_scatter(1)
    plsc.subcore_barrier()
    pltpu.sync_copy(acc.at[pl.ds(s * RPS, RPS)],
                    out.at[c, pl.ds(s * RPS, RPS)])


def _sc_edge(t2, ee, di, si):
    mesh = plsc.VectorSubcoreMesh(core_axis_name="c", subcore_axis_name="s",
                                  num_cores=NC, num_subcores=NS)
    fn = pl.kernel(
        _sc_edge_body,
        out_type=jax.ShapeDtypeStruct((NC, NP, F), jnp.float32),
        mesh=mesh,
        scratch_types=[
            pltpu.VMEM((2 * BE,), jnp.int32),
            pltpu.VMEM((2 * BE,), jnp.int32),
            pltpu.VMEM((BE,), jnp.int32),
            pltpu.VMEM((BE,), jnp.int32),
            pltpu.VMEM((2 * BE, 2 * F), jnp.float32),
            pltpu.VMEM((2 * BE, 2 * F), jnp.float32),
            pltpu.VMEM((BE, 2 * F), jnp.float32),
            pltpu.VMEM((BE, 2 * F), jnp.float32),
            pltpu.VMEM((BE, F), jnp.float32),
            pltpu.VMEM((BE, F), jnp.float32),
            pltpu.VMEM((ZB, F), jnp.float32),
            pltpu.VMEM_SHARED((NP, F), jnp.float32),
        ] + [pltpu.SemaphoreType.DMA] * 10,
    )
    return fn(t2, ee, di, si)


@jax.jit
def kernel(x, edge_index, edge_attr, batch,
           Wf1, bf1, Ws1, bs1, Wf2, bf2, Ws2, bs2, Wo, bo):
    dsti = edge_index[1]
    srci = edge_index[0]

    w1 = jnp.concatenate([Wf1[:F], Ws1[:F], Wf1[F:2 * F], Ws1[F:2 * F]],
                         axis=1)
    b1 = jnp.concatenate(
        [bf1, bs1, jnp.zeros((2 * F,), jnp.float32)]).reshape(1, 4 * F)
    w2 = jnp.concatenate([Wf2[:F], Ws2[:F], Wf2[F:2 * F], Ws2[F:2 * F]],
                         axis=1)
    b2 = jnp.concatenate(
        [bf2, bs2, jnp.zeros((2 * F,), jnp.float32)]).reshape(1, 4 * F)
    eye8 = jnp.eye(EPK, dtype=jnp.float32)
    we1 = jnp.kron(eye8, jnp.concatenate([Wf1[2 * F:], Ws1[2 * F:]], axis=1))
    we2 = jnp.kron(eye8, jnp.concatenate([Wf2[2 * F:], Ws2[2 * F:]], axis=1))
    ea8 = edge_attr.reshape(E // EPK, EPK * D)

    ee1 = _edge_proj(ea8, we1)
    t1 = _node_proj1(x, w1, b1)
    aggp1 = _sc_edge(t1.reshape(2 * NP, 2 * F), ee1, dsti, srci)
    ee2 = _edge_proj(ea8, we2)
    h1, t2 = _node_proj2(x, aggp1, w2, b2)
    aggp2 = _sc_edge(t2.reshape(2 * NP, 2 * F), ee2, dsti, srci)
    out = _pool(h1, aggp2, batch.reshape(N, 1), Wo, bo.reshape(1, 1))
    return out


# speedup vs baseline: 1.2677x; 1.2677x over previous
import jax
import jax.numpy as jnp
from jax import lax
from jax.experimental import pallas as pl
from jax.experimental.pallas import tpu as pltpu
from jax.experimental.pallas import tpu_sc as plsc

N = 10000
E = 320000
F = 128
D = 4
G = 256

NC = 2
NS = 16
NW = NC * NS
EW = E // NW
BE = 16
NB = EW // BE
ZB = 64
NP = 10240
RPS = NP // NS

_LOG1P_C = (
    0.00014158017492754693, 0.995426661775425, -0.4640707011025748,
    0.21640858368174304, -0.05486231128931281,
)

BN = 400
BEP = 3200


def _proj1_body(x_ref, w_ref, b_ref, t_ref):
    t_ref[0] = jnp.dot(x_ref[...], w_ref[...],
                       preferred_element_type=jnp.float32) + b_ref[...]


def _proj2_body(x_ref, a_ref, w_ref, b_ref, h_ref, t_ref):
    h = x_ref[...] + a_ref[0] + a_ref[1]
    h_ref[...] = h
    t_ref[0] = jnp.dot(h, w_ref[...],
                       preferred_element_type=jnp.float32) + b_ref[...]


def _edge_proj_body(ea_ref, we_ref, e_ref):
    e_ref[...] = jnp.dot(ea_ref[...], we_ref[...],
                         preferred_element_type=jnp.float32)


def _pool_body(h_ref, a_ref, b_ref, wo_ref, bo_ref, o_ref, acc_ref):
    i = pl.program_id(0)

    @pl.when(i == 0)
    def _():
        acc_ref[...] = jnp.zeros_like(acc_ref)

    h2 = h_ref[...] + a_ref[0] + a_ref[1]
    oh = (b_ref[...] == lax.broadcasted_iota(jnp.int32, (BN, G), 1)
          ).astype(jnp.float32)
    acc_ref[...] += lax.dot_general(oh, h2, (((0,), (0,)), ((), ())),
                                    preferred_element_type=jnp.float32)

    @pl.when(i == pl.num_programs(0) - 1)
    def _():
        o_ref[...] = jnp.dot(acc_ref[...], wo_ref[...],
                             preferred_element_type=jnp.float32) + bo_ref[...]


def _node_proj1(x, w, b):
    return pl.pallas_call(
        _proj1_body,
        grid=(N // BN, 2),
        in_specs=[
            pl.BlockSpec((BN, F), lambda i, j: (i, 0)),
            pl.BlockSpec((F, 2 * F), lambda i, j: (0, j)),
            pl.BlockSpec((1, 2 * F), lambda i, j: (0, j)),
        ],
        out_specs=pl.BlockSpec((1, BN, 2 * F), lambda i, j: (j, i, 0)),
        out_shape=jax.ShapeDtypeStruct((2, NP, 2 * F), jnp.float32),
    )(x, w, b)


def _node_proj2(x, aggp, w, b):
    return pl.pallas_call(
        _proj2_body,
        grid=(N // BN, 2),
        in_specs=[
            pl.BlockSpec((BN, F), lambda i, j: (i, 0)),
            pl.BlockSpec((2, BN, F), lambda i, j: (0, i, 0)),
            pl.BlockSpec((F, 2 * F), lambda i, j: (0, j)),
            pl.BlockSpec((1, 2 * F), lambda i, j: (0, j)),
        ],
        out_specs=[
            pl.BlockSpec((BN, F), lambda i, j: (i, 0)),
            pl.BlockSpec((1, BN, 2 * F), lambda i, j: (j, i, 0)),
        ],
        out_shape=[
            jax.ShapeDtypeStruct((N, F), jnp.float32),
            jax.ShapeDtypeStruct((2, NP, 2 * F), jnp.float32),
        ],
    )(x, aggp, w, b)


def _edge_proj(ea, we):
    return pl.pallas_call(
        _edge_proj_body,
        grid=(E // BEP,),
        in_specs=[
            pl.BlockSpec((BEP, D), lambda i: (i, 0)),
            pl.BlockSpec((D, 2 * F), lambda i: (0, 0)),
        ],
        out_specs=pl.BlockSpec((BEP, 2 * F), lambda i: (i, 0)),
        out_shape=jax.ShapeDtypeStruct((E, 2 * F), jnp.float32),
    )(ea, we)


def _pool(h, aggp, batch2d, wo, bo):
    return pl.pallas_call(
        _pool_body,
        grid=(N // BN,),
        in_specs=[
            pl.BlockSpec((BN, F), lambda i: (i, 0)),
            pl.BlockSpec((2, BN, F), lambda i: (0, i, 0)),
            pl.BlockSpec((BN, 1), lambda i: (i, 0)),
            pl.BlockSpec((F, 1), lambda i: (0, 0)),
            pl.BlockSpec((1, 1), lambda i: (0, 0)),
        ],
        out_specs=pl.BlockSpec((G, 1), lambda i: (0, 0)),
        out_shape=jax.ShapeDtypeStruct((G, 1), jnp.float32),
        scratch_shapes=[pltpu.VMEM((G, F), jnp.float32)],
    )(h, aggp, batch2d, wo, bo)


def _msg_edge(g, geb, mbb, e):
    sls = [pl.ds(ch * 16, 16) for ch in range(8)]
    sl2s = [pl.ds(F + ch * 16, 16) for ch in range(8)]
    a = [g[e, s] + g[BE + e, s] + geb[e, s] for s in sls]
    b2 = [g[e, s] + g[BE + e, s] + geb[e, s] for s in sl2s]
    u = [jnp.exp(-x) for x in a]
    r = [1.0 / (1.0 + x) for x in u]
    t = [jnp.exp(jnp.minimum(x, -x)) for x in b2]
    lp = [jnp.full((16,), _LOG1P_C[-1], jnp.float32)] * 8
    for co in _LOG1P_C[-2::-1]:
        lp = [p * x + co for p, x in zip(lp, t)]
    sp = [jnp.maximum(x, 0.0) + p for x, p in zip(b2, lp)]
    for ch in range(8):
        mbb[e, sls[ch]] = r[ch] * sp[ch]


def _sc_edge_body(t2, ee, di, si, out,
                  comb0, comb1, sdx0, sdx1, gb0, gb1, ge0, ge1, mb0, mb1,
                  zb, acc,
                  semg0, semg1, seme0, seme1, semi0, semi1,
                  semx0, semx1, sems0, sems1):
    c = lax.axis_index("c")
    s = lax.axis_index("s")
    w = s * NC + c
    comb = (comb0, comb1)
    sdx = (sdx0, sdx1)
    gb = (gb0, gb1)
    ge = (ge0, ge1)
    mb = (mb0, mb1)
    semg = (semg0, semg1)
    seme = (seme0, seme1)
    semi = (semi0, semi1)
    semx = (semx0, semx1)
    sems = (sems0, sems1)

    def _zrow(e, carry):
        for ch in range(8):
            zb[e, pl.ds(ch * 16, 16)] = jnp.zeros((16,), jnp.float32)
        return carry

    lax.fori_loop(0, ZB, _zrow, 0)
    for j in range(RPS // ZB):
        pltpu.sync_copy(zb, acc.at[pl.ds(s * RPS + j * ZB, ZB)])
    plsc.subcore_barrier()

    def _ibase(b):
        return w * EW + b * BE

    def _issue_idx(b, sl):
        base = _ibase(b)
        pltpu.async_copy(di.at[pl.ds(base, BE)],
                         comb[sl].at[pl.ds(0, BE)], semi[sl])
        pltpu.async_copy(si.at[pl.ds(base, BE)],
                         comb[sl].at[pl.ds(BE, BE)], semi[sl])

    def _wait_idx(b, sl):
        base = _ibase(b)
        pltpu.make_async_copy(di.at[pl.ds(base, BE)],
                              comb[sl].at[pl.ds(0, BE)], semi[sl]).wait()
        pltpu.make_async_copy(si.at[pl.ds(base, BE)],
                              comb[sl].at[pl.ds(BE, BE)], semi[sl]).wait()
        comb[sl][pl.ds(BE, BE)] = comb[sl][pl.ds(BE, BE)] + NP

    def _issue_gather(b, sl):
        pltpu.async_copy(t2.at[comb[sl]], gb[sl], semg[sl])
        pltpu.async_copy(ee.at[pl.ds(_ibase(b), BE)], ge[sl], seme[sl])

    def _wait_gather(b, sl):
        pltpu.make_async_copy(t2.at[comb[sl]], gb[sl], semg[sl]).wait()
        pltpu.make_async_copy(ee.at[pl.ds(_ibase(b), BE)], ge[sl],
                              seme[sl]).wait()

    def _wait_scatter(sl):
        pltpu.make_async_copy(mb[sl], acc.at[sdx[sl]], sems[sl]).wait()

    pltpu.sync_copy(di.at[pl.ds(_ibase(0), BE)], comb0.at[pl.ds(0, BE)])
    pltpu.sync_copy(si.at[pl.ds(_ibase(0), BE)], comb0.at[pl.ds(BE, BE)])
    comb0[pl.ds(BE, BE)] = comb0[pl.ds(BE, BE)] + NP
    _issue_gather(0, 0)
    _issue_idx(1, 1)

    def _batch(b, sl):
        bt = jnp.int32(b)
        nsl = 1 - sl

        @pl.when(bt + 1 < NB)
        def _():
            _wait_idx(b + 1, nsl)
            _issue_gather(b + 1, nsl)

        _wait_gather(b, sl)

        @pl.when(bt + 2 < NB)
        def _():
            _issue_idx(b + 2, sl)

        @pl.when(bt >= 2)
        def _():
            _wait_scatter(sl)

        pltpu.async_copy(di.at[pl.ds(_ibase(b), BE)], sdx[sl], semx[sl])

        @plsc.parallel_loop(0, BE, unroll=4)
        def _edge(e):
            _msg_edge(gb[sl], ge[sl], mb[sl], e)

        pltpu.make_async_copy(di.at[pl.ds(_ibase(b), BE)], sdx[sl],
                              semx[sl]).wait()
        pltpu.async_copy(mb[sl], acc.at[sdx[sl]], sems[sl], add=True)

    def _super(o, carry):
        for sl in (0, 1):
            _batch(o * 2 + sl, sl)
        return carry

    lax.fori_loop(0, NB // 2, _super, 0)
    if NB % 2:
        _batch(NB - 1, 0)
    _wait_scatter(0)
    _wait_scatter(1)
    plsc.subcore_barrier()
    pltpu.sync_copy(acc.at[pl.ds(s * RPS, RPS)],
                    out.at[c, pl.ds(s * RPS, RPS)])


def _sc_edge(t2, ee, di, si):
    mesh = plsc.VectorSubcoreMesh(core_axis_name="c", subcore_axis_name="s",
                                  num_cores=NC, num_subcores=NS)
    fn = pl.kernel(
        _sc_edge_body,
        out_type=jax.ShapeDtypeStruct((NC, NP, F), jnp.float32),
        mesh=mesh,
        scratch_types=[
            pltpu.VMEM((2 * BE,), jnp.int32),
            pltpu.VMEM((2 * BE,), jnp.int32),
            pltpu.VMEM((BE,), jnp.int32),
            pltpu.VMEM((BE,), jnp.int32),
            pltpu.VMEM((2 * BE, 2 * F), jnp.float32),
            pltpu.VMEM((2 * BE, 2 * F), jnp.float32),
            pltpu.VMEM((BE, 2 * F), jnp.float32),
            pltpu.VMEM((BE, 2 * F), jnp.float32),
            pltpu.VMEM((BE, F), jnp.float32),
            pltpu.VMEM((BE, F), jnp.float32),
            pltpu.VMEM((ZB, F), jnp.float32),
            pltpu.VMEM_SHARED((NP, F), jnp.float32),
        ] + [pltpu.SemaphoreType.DMA] * 10,
    )
    return fn(t2, ee, di, si)


@jax.jit
def kernel(x, edge_index, edge_attr, batch,
           Wf1, bf1, Ws1, bs1, Wf2, bf2, Ws2, bs2, Wo, bo):
    dsti = edge_index[1]
    srci = edge_index[0]

    w1 = jnp.concatenate([Wf1[:F], Ws1[:F], Wf1[F:2 * F], Ws1[F:2 * F]],
                         axis=1)
    b1 = jnp.concatenate(
        [bf1, bs1, jnp.zeros((2 * F,), jnp.float32)]).reshape(1, 4 * F)
    w2 = jnp.concatenate([Wf2[:F], Ws2[:F], Wf2[F:2 * F], Ws2[F:2 * F]],
                         axis=1)
    b2 = jnp.concatenate(
        [bf2, bs2, jnp.zeros((2 * F,), jnp.float32)]).reshape(1, 4 * F)
    we1 = jnp.concatenate([Wf1[2 * F:], Ws1[2 * F:]], axis=1)
    we2 = jnp.concatenate([Wf2[2 * F:], Ws2[2 * F:]], axis=1)

    ee1 = _edge_proj(edge_attr, we1)
    t1 = _node_proj1(x, w1, b1)
    aggp1 = _sc_edge(t1.reshape(2 * NP, 2 * F), ee1, dsti, srci)
    ee2 = _edge_proj(edge_attr, we2)
    h1, t2 = _node_proj2(x, aggp1, w2, b2)
    aggp2 = _sc_edge(t2.reshape(2 * NP, 2 * F), ee2, dsti, srci)
    out = _pool(h1, aggp2, batch.reshape(N, 1), Wo, bo.reshape(1, 1))
    return out


# speedup vs baseline: 1.2743x; 1.0052x over previous
import jax
import jax.numpy as jnp
from jax import lax
from jax.experimental import pallas as pl
from jax.experimental.pallas import tpu as pltpu
from jax.experimental.pallas import tpu_sc as plsc

N = 10000
E = 320000
F = 128
D = 4
G = 256

NC = 2
NS = 16
NW = NC * NS
EW = E // NW
BE = 16
NB = EW // BE
ZB = 64
NP = 10240
RPS = NP // NS

_LOG1P_C = (
    0.00014158017492754693, 0.995426661775425, -0.4640707011025748,
    0.21640858368174304, -0.05486231128931281,
)

BN = 400
BEP = 6400


def _proj1_body(x_ref, w_ref, b_ref, t_ref):
    t_ref[0] = jnp.dot(x_ref[...], w_ref[...],
                       preferred_element_type=jnp.float32) + b_ref[...]


def _proj2_body(x_ref, a_ref, w_ref, b_ref, h_ref, t_ref):
    h = x_ref[...] + a_ref[0] + a_ref[1]
    h_ref[...] = h
    t_ref[0] = jnp.dot(h, w_ref[...],
                       preferred_element_type=jnp.float32) + b_ref[...]


def _edge_proj_body(ea_ref, we_ref, e_ref):
    e_ref[...] = jnp.dot(ea_ref[...], we_ref[...],
                         preferred_element_type=jnp.float32)


def _pool_body(h_ref, a_ref, b_ref, wo_ref, bo_ref, o_ref, acc_ref):
    i = pl.program_id(0)

    @pl.when(i == 0)
    def _():
        acc_ref[...] = jnp.zeros_like(acc_ref)

    h2 = h_ref[...] + a_ref[0] + a_ref[1]
    oh = (b_ref[...] == lax.broadcasted_iota(jnp.int32, (BN, G), 1)
          ).astype(jnp.float32)
    acc_ref[...] += lax.dot_general(oh, h2, (((0,), (0,)), ((), ())),
                                    preferred_element_type=jnp.float32)

    @pl.when(i == pl.num_programs(0) - 1)
    def _():
        o_ref[...] = jnp.dot(acc_ref[...], wo_ref[...],
                             preferred_element_type=jnp.float32) + bo_ref[...]


def _node_proj1(x, w, b):
    return pl.pallas_call(
        _proj1_body,
        grid=(N // BN, 2),
        in_specs=[
            pl.BlockSpec((BN, F), lambda i, j: (i, 0)),
            pl.BlockSpec((F, 2 * F), lambda i, j: (0, j)),
            pl.BlockSpec((1, 2 * F), lambda i, j: (0, j)),
        ],
        out_specs=pl.BlockSpec((1, BN, 2 * F), lambda i, j: (j, i, 0)),
        out_shape=jax.ShapeDtypeStruct((2, NP, 2 * F), jnp.float32),
    )(x, w, b)


def _node_proj2(x, aggp, w, b):
    return pl.pallas_call(
        _proj2_body,
        grid=(N // BN, 2),
        in_specs=[
            pl.BlockSpec((BN, F), lambda i, j: (i, 0)),
            pl.BlockSpec((2, BN, F), lambda i, j: (0, i, 0)),
            pl.BlockSpec((F, 2 * F), lambda i, j: (0, j)),
            pl.BlockSpec((1, 2 * F), lambda i, j: (0, j)),
        ],
        out_specs=[
            pl.BlockSpec((BN, F), lambda i, j: (i, 0)),
            pl.BlockSpec((1, BN, 2 * F), lambda i, j: (j, i, 0)),
        ],
        out_shape=[
            jax.ShapeDtypeStruct((N, F), jnp.float32),
            jax.ShapeDtypeStruct((2, NP, 2 * F), jnp.float32),
        ],
    )(x, aggp, w, b)


def _edge_proj(ea, we):
    return pl.pallas_call(
        _edge_proj_body,
        grid=(E // BEP,),
        in_specs=[
            pl.BlockSpec((BEP, D), lambda i: (i, 0)),
            pl.BlockSpec((D, 2 * F), lambda i: (0, 0)),
        ],
        out_specs=pl.BlockSpec((BEP, 2 * F), lambda i: (i, 0)),
        out_shape=jax.ShapeDtypeStruct((E, 2 * F), jnp.float32),
    )(ea, we)


def _pool(h, aggp, batch2d, wo, bo):
    return pl.pallas_call(
        _pool_body,
        grid=(N // BN,),
        in_specs=[
            pl.BlockSpec((BN, F), lambda i: (i, 0)),
            pl.BlockSpec((2, BN, F), lambda i: (0, i, 0)),
            pl.BlockSpec((BN, 1), lambda i: (i, 0)),
            pl.BlockSpec((F, 1), lambda i: (0, 0)),
            pl.BlockSpec((1, 1), lambda i: (0, 0)),
        ],
        out_specs=pl.BlockSpec((G, 1), lambda i: (0, 0)),
        out_shape=jax.ShapeDtypeStruct((G, 1), jnp.float32),
        scratch_shapes=[pltpu.VMEM((G, F), jnp.float32)],
    )(h, aggp, batch2d, wo, bo)


def _msg_edge(g, geb, mbb, e):
    sls = [pl.ds(ch * 16, 16) for ch in range(8)]
    sl2s = [pl.ds(F + ch * 16, 16) for ch in range(8)]
    a = [g[e, s] + g[BE + e, s] + geb[e, s] for s in sls]
    b2 = [g[e, s] + g[BE + e, s] + geb[e, s] for s in sl2s]
    u = [jnp.exp(-x) for x in a]
    r = [1.0 / (1.0 + x) for x in u]
    t = [jnp.exp(jnp.minimum(x, -x)) for x in b2]
    lp = [jnp.full((16,), _LOG1P_C[-1], jnp.float32)] * 8
    for co in _LOG1P_C[-2::-1]:
        lp = [p * x + co for p, x in zip(lp, t)]
    sp = [jnp.maximum(x, 0.0) + p for x, p in zip(b2, lp)]
    for ch in range(8):
        mbb[e, sls[ch]] = r[ch] * sp[ch]


def _sc_edge_body(t2, ee, di, si, out,
                  comb0, comb1, sdx0, sdx1, gb0, gb1, ge0, ge1, mb0, mb1,
                  zb, acc,
                  semg0, semg1, seme0, seme1, semi0, semi1,
                  semx0, semx1, sems0, sems1):
    c = lax.axis_index("c")
    s = lax.axis_index("s")
    w = s * NC + c
    comb = (comb0, comb1)
    sdx = (sdx0, sdx1)
    gb = (gb0, gb1)
    ge = (ge0, ge1)
    mb = (mb0, mb1)
    semg = (semg0, semg1)
    seme = (seme0, seme1)
    semi = (semi0, semi1)
    semx = (semx0, semx1)
    sems = (sems0, sems1)

    def _zrow(e, carry):
        for ch in range(8):
            zb[e, pl.ds(ch * 16, 16)] = jnp.zeros((16,), jnp.float32)
        return carry

    lax.fori_loop(0, ZB, _zrow, 0)
    for j in range(RPS // ZB):
        pltpu.sync_copy(zb, acc.at[pl.ds(s * RPS + j * ZB, ZB)])
    plsc.subcore_barrier()

    def _ibase(b):
        return w * EW + b * BE

    def _issue_idx(b, sl):
        base = _ibase(b)
        pltpu.async_copy(di.at[pl.ds(base, BE)],
                         comb[sl].at[pl.ds(0, BE)], semi[sl])
        pltpu.async_copy(si.at[pl.ds(base, BE)],
                         comb[sl].at[pl.ds(BE, BE)], semi[sl])

    def _wait_idx(b, sl):
        base = _ibase(b)
        pltpu.make_async_copy(di.at[pl.ds(base, BE)],
                              comb[sl].at[pl.ds(0, BE)], semi[sl]).wait()
        pltpu.make_async_copy(si.at[pl.ds(base, BE)],
                              comb[sl].at[pl.ds(BE, BE)], semi[sl]).wait()
        comb[sl][pl.ds(BE, BE)] = comb[sl][pl.ds(BE, BE)] + NP

    def _issue_gather(b, sl):
        pltpu.async_copy(t2.at[comb[sl]], gb[sl], semg[sl])
        pltpu.async_copy(ee.at[pl.ds(_ibase(b), BE)], ge[sl], seme[sl])

    def _wait_gather(b, sl):
        pltpu.make_async_copy(t2.at[comb[sl]], gb[sl], semg[sl]).wait()
        pltpu.make_async_copy(ee.at[pl.ds(_ibase(b), BE)], ge[sl],
                              seme[sl]).wait()

    def _wait_scatter(sl):
        pltpu.make_async_copy(mb[sl], acc.at[sdx[sl]], sems[sl]).wait()

    pltpu.sync_copy(di.at[pl.ds(_ibase(0), BE)], comb0.at[pl.ds(0, BE)])
    pltpu.sync_copy(si.at[pl.ds(_ibase(0), BE)], comb0.at[pl.ds(BE, BE)])
    comb0[pl.ds(BE, BE)] = comb0[pl.ds(BE, BE)] + NP
    _issue_gather(0, 0)
    _issue_idx(1, 1)

    def _batch(b, sl):
        bt = jnp.int32(b)
        nsl = 1 - sl

        @pl.when(bt + 1 < NB)
        def _():
            _wait_idx(b + 1, nsl)
            _issue_gather(b + 1, nsl)

        _wait_gather(b, sl)

        @pl.when(bt + 2 < NB)
        def _():
            _issue_idx(b + 2, sl)

        @pl.when(bt >= 2)
        def _():
            _wait_scatter(sl)

        pltpu.async_copy(di.at[pl.ds(_ibase(b), BE)], sdx[sl], semx[sl])

        @plsc.parallel_loop(0, BE, unroll=4)
        def _edge(e):
            _msg_edge(gb[sl], ge[sl], mb[sl], e)

        pltpu.make_async_copy(di.at[pl.ds(_ibase(b), BE)], sdx[sl],
                              semx[sl]).wait()
        pltpu.async_copy(mb[sl], acc.at[sdx[sl]], sems[sl], add=True)

    def _super(o, carry):
        for sl in (0, 1):
            _batch(o * 2 + sl, sl)
        return carry

    lax.fori_loop(0, NB // 2, _super, 0)
    if NB % 2:
        _batch(NB - 1, 0)
    _wait_scatter(0)
    _wait_scatter(1)
    plsc.subcore_barrier()
    pltpu.sync_copy(acc.at[pl.ds(s * RPS, RPS)],
                    out.at[c, pl.ds(s * RPS, RPS)])


def _sc_edge(t2, ee, di, si):
    mesh = plsc.VectorSubcoreMesh(core_axis_name="c", subcore_axis_name="s",
                                  num_cores=NC, num_subcores=NS)
    fn = pl.kernel(
        _sc_edge_body,
        out_type=jax.ShapeDtypeStruct((NC, NP, F), jnp.float32),
        mesh=mesh,
        scratch_types=[
            pltpu.VMEM((2 * BE,), jnp.int32),
            pltpu.VMEM((2 * BE,), jnp.int32),
            pltpu.VMEM((BE,), jnp.int32),
            pltpu.VMEM((BE,), jnp.int32),
            pltpu.VMEM((2 * BE, 2 * F), jnp.float32),
            pltpu.VMEM((2 * BE, 2 * F), jnp.float32),
            pltpu.VMEM((BE, 2 * F), jnp.float32),
            pltpu.VMEM((BE, 2 * F), jnp.float32),
            pltpu.VMEM((BE, F), jnp.float32),
            pltpu.VMEM((BE, F), jnp.float32),
            pltpu.VMEM((ZB, F), jnp.float32),
            pltpu.VMEM_SHARED((NP, F), jnp.float32),
        ] + [pltpu.SemaphoreType.DMA] * 10,
    )
    return fn(t2, ee, di, si)


@jax.jit
def kernel(x, edge_index, edge_attr, batch,
           Wf1, bf1, Ws1, bs1, Wf2, bf2, Ws2, bs2, Wo, bo):
    dsti = edge_index[1]
    srci = edge_index[0]

    w1 = jnp.concatenate([Wf1[:F], Ws1[:F], Wf1[F:2 * F], Ws1[F:2 * F]],
                         axis=1)
    b1 = jnp.concatenate(
        [bf1, bs1, jnp.zeros((2 * F,), jnp.float32)]).reshape(1, 4 * F)
    w2 = jnp.concatenate([Wf2[:F], Ws2[:F], Wf2[F:2 * F], Ws2[F:2 * F]],
                         axis=1)
    b2 = jnp.concatenate(
        [bf2, bs2, jnp.zeros((2 * F,), jnp.float32)]).reshape(1, 4 * F)
    we1 = jnp.concatenate([Wf1[2 * F:], Ws1[2 * F:]], axis=1)
    we2 = jnp.concatenate([Wf2[2 * F:], Ws2[2 * F:]], axis=1)

    ee1 = _edge_proj(edge_attr, we1)
    t1 = _node_proj1(x, w1, b1)
    aggp1 = _sc_edge(t1.reshape(2 * NP, 2 * F), ee1, dsti, srci)
    ee2 = _edge_proj(edge_attr, we2)
    h1, t2 = _node_proj2(x, aggp1, w2, b2)
    aggp2 = _sc_edge(t2.reshape(2 * NP, 2 * F), ee2, dsti, srci)
    out = _pool(h1, aggp2, batch.reshape(N, 1), Wo, bo.reshape(1, 1))
    return out
